# baseline (speedup 1.0000x reference)
import functools

import jax
import jax.numpy as jnp
from jax.experimental import pallas as pl
from jax.experimental.pallas import tpu as pltpu

_IN_FEATURES = 10
_OUT_FEATURES = 5
_CORES = 1
_STEPS = 32
_IN_SLOTS = 4
_OUT_SLOTS = 3


def _linear_body(w_ref, b_ref, xT_hbm, oT_hbm, xbuf, ybuf, in_sem, out_sem,
                 *, tb, steps):
    core = pl.program_id(0)
    base = core * steps * tb

    def in_copy(s, slot):
        return pltpu.make_async_copy(
            xT_hbm.at[:, pl.ds(base + s * tb, tb)], xbuf.at[slot],
            in_sem.at[slot])

    def out_copy(s, slot):
        return pltpu.make_async_copy(
            ybuf.at[slot], oT_hbm.at[:, pl.ds(base + s * tb, tb)],
            out_sem.at[slot])

    for s in range(min(_IN_SLOTS - 1, steps)):
        in_copy(s, s % _IN_SLOTS).start()
    for s in range(steps):
        isl = s % _IN_SLOTS
        osl = s % _OUT_SLOTS
        if s + _IN_SLOTS - 1 < steps:
            in_copy(s + _IN_SLOTS - 1, (s + _IN_SLOTS - 1) % _IN_SLOTS).start()
        in_copy(s, isl).wait()
        if s >= _OUT_SLOTS:
            out_copy(s - _OUT_SLOTS, osl).wait()
        y = jnp.dot(w_ref[...], xbuf[isl],
                    preferred_element_type=jnp.float32)
        ybuf[osl] = (y + b_ref[...]).astype(ybuf.dtype)
        out_copy(s, osl).start()
    for s in range(max(steps - _OUT_SLOTS, 0), steps):
        out_copy(s, s % _OUT_SLOTS).wait()


def kernel(x, weight, bias):
    orig_B = x.shape[0]
    chunk = _CORES * _STEPS * 128
    B = orig_B
    if B % chunk != 0:
        pad = chunk - B % chunk
        x = jnp.pad(x, ((0, pad), (0, 0)))
        B = B + pad
    tb = B // (_CORES * _STEPS)

    xT = x.T
    b2 = bias.reshape(_OUT_FEATURES, 1)

    oT = pl.pallas_call(
        functools.partial(_linear_body, tb=tb, steps=_STEPS),
        out_shape=jax.ShapeDtypeStruct((_OUT_FEATURES, B), x.dtype),
        grid=(_CORES,),
        in_specs=[
            pl.BlockSpec((_OUT_FEATURES, _IN_FEATURES), lambda i: (0, 0)),
            pl.BlockSpec((_OUT_FEATURES, 1), lambda i: (0, 0)),
            pl.BlockSpec(memory_space=pltpu.MemorySpace.HBM),
        ],
        out_specs=pl.BlockSpec(memory_space=pltpu.MemorySpace.HBM),
        scratch_shapes=[
            pltpu.VMEM((_IN_SLOTS, _IN_FEATURES, tb), jnp.float32),
            pltpu.VMEM((_OUT_SLOTS, _OUT_FEATURES, tb), jnp.float32),
            pltpu.SemaphoreType.DMA((_IN_SLOTS,)),
            pltpu.SemaphoreType.DMA((_OUT_SLOTS,)),
        ],
        compiler_params=pltpu.CompilerParams(
            dimension_semantics=("parallel",),
            vmem_limit_bytes=64 * 1024 * 1024,
        ),
    )(weight, b2, xT)

    return oT.T[:orig_B]
